# baseline (speedup 1.0000x reference)
import jax
import jax.numpy as jnp
from jax.experimental import pallas as pl
from jax.experimental.pallas import tpu as pltpu

_EMBED_DIM = 128
_RULES_SIZE = 1000
_ROWS_PER_BLOCK = 8


def _fused_kernel(ident_ref, row_ref, w_ref, b_ref, out_ref):
    r = ident_ref[0] % _ROWS_PER_BLOCK
    row = row_ref[pl.ds(r, 1), :]
    acc = jnp.dot(row, w_ref[...], preferred_element_type=jnp.float32)
    out_ref[...] = jnp.maximum(acc + b_ref[...], 0.0)


def kernel(ident, table, W, b):
    ident = ident.astype(jnp.int32)
    grid_spec = pltpu.PrefetchScalarGridSpec(
        num_scalar_prefetch=1,
        grid=(1,),
        in_specs=[
            pl.BlockSpec(
                (_ROWS_PER_BLOCK, _EMBED_DIM),
                lambda i, ident_ref: (ident_ref[0] // _ROWS_PER_BLOCK, 0),
            ),
            pl.BlockSpec((_EMBED_DIM, _RULES_SIZE), lambda i, ident_ref: (0, 0)),
            pl.BlockSpec((1, _RULES_SIZE), lambda i, ident_ref: (0, 0)),
        ],
        out_specs=pl.BlockSpec((1, _RULES_SIZE), lambda i, ident_ref: (0, 0)),
    )
    return pl.pallas_call(
        _fused_kernel,
        grid_spec=grid_spec,
        out_shape=jax.ShapeDtypeStruct((1, _RULES_SIZE), jnp.float32),
    )(ident, table, W, b.reshape(1, _RULES_SIZE))
